# baseline (speedup 1.0000x reference)
import functools

import jax
import jax.numpy as jnp
from jax import lax
from jax.experimental import pallas as pl
from jax.experimental.pallas import tpu as pltpu
from jax.experimental.pallas import tpu_sc as plsc

N = 10000
E = 320000
F = 128
NC = 2
NS = 16
NW = NC * NS
K = 125
HALF = 40
ROWS = E // K
CPW = ROWS // NW
NPT = 624
TAIL = N - NS * NPT
DW = 16

_mesh = plsc.VectorSubcoreMesh(core_axis_name="c", subcore_axis_name="s")


EPW = E // NW
HR = N // DW


def _deg_body(src_hbm, iota_hbm, deg_hbm, idx_v, hist_v, iota_v, acc_sh):
    c = lax.axis_index("c")
    s = lax.axis_index("s")
    wid = c * NS + s

    pltpu.sync_copy(src_hbm.at[pl.ds(wid * EPW, EPW)], idx_v)
    pltpu.sync_copy(iota_hbm, iota_v)

    zeros16 = jnp.zeros((16,), jnp.float32)

    def zero_row(i, _):
        hist_v[i, :] = zeros16
        return 0
    lax.fori_loop(0, HR, zero_row, 0)

    @pl.when(s == 0)
    def _():
        pltpu.sync_copy(hist_v, acc_sh)
    plsc.subcore_barrier()

    ones16 = jnp.ones((16,), jnp.float32)

    def body(i, _):
        idx = idx_v[pl.ds(i * 16, 16)]
        plsc.addupdate_scatter(hist_v, [idx >> 4, idx & 15], ones16)
        return 0
    lax.fori_loop(0, EPW // 16, body, 0)

    for z in range(HR // K):
        pltpu.sync_copy(hist_v.at[pl.ds(z * K, K)],
                        acc_sh.at[iota_v.at[z]], add=True)
    plsc.subcore_barrier()

    @pl.when(s == 0)
    def _():
        pltpu.sync_copy(acc_sh, deg_hbm.at[c])


_deg_call = functools.partial(
    pl.kernel,
    out_type=jax.ShapeDtypeStruct((NC, HR, DW), jnp.float32),
    mesh=_mesh,
    compiler_params=pltpu.CompilerParams(needs_layout_passes=False),
    scratch_types=[
        pltpu.VMEM((EPW,), jnp.int32),
        pltpu.VMEM((HR, DW), jnp.float32),
        pltpu.VMEM((HR // K, K), jnp.int32),
        pltpu.VMEM_SHARED((HR, DW), jnp.float32),
    ],
)(_deg_body)


def _mm_body(x_ref, w_ref, dg_ref, hs_ref):
    h = jnp.dot(x_ref[...], w_ref[...], preferred_element_type=jnp.float32)
    d = dg_ref[0] + dg_ref[1]
    hs_ref[...] = h * lax.rsqrt(d)


def _agg_body(hs_hbm, src_hbm, dst_hbm, p_hbm, sidx, didx, rows0, rows1,
              sem0, sem1, acc_sh):
    c = lax.axis_index("c")
    s = lax.axis_index("s")
    wid = c * NS + s
    row0 = wid * CPW

    pltpu.sync_copy(hs_hbm.at[pl.ds(s * NPT, NPT)],
                    acc_sh.at[pl.ds(s * NPT, NPT)])

    @pl.when(s == 0)
    def _():
        pltpu.sync_copy(hs_hbm.at[pl.ds(NS * NPT, TAIL)],
                        acc_sh.at[pl.ds(NS * NPT, TAIL)])
    plsc.subcore_barrier()

    for h in range(CPW // HALF):
        pltpu.sync_copy(src_hbm.at[pl.ds(row0 + h * HALF, HALF)], sidx)
        pltpu.sync_copy(dst_hbm.at[pl.ds(row0 + h * HALF, HALF)], didx)
        pltpu.async_copy(hs_hbm.at[sidx.at[0]], rows0, sem0)

        def body(i, _):
            j0 = 2 * i
            pltpu.async_copy(hs_hbm.at[sidx.at[j0 + 1]], rows1, sem1)
            pltpu.make_async_copy(hs_hbm.at[sidx.at[j0]], rows0, sem0).wait()
            pltpu.sync_copy(rows0, acc_sh.at[didx.at[j0]], add=True)

            @pl.when(j0 + 2 < HALF)
            def _():
                pltpu.async_copy(hs_hbm.at[sidx.at[j0 + 2]], rows0, sem0)
            pltpu.make_async_copy(hs_hbm.at[sidx.at[j0 + 1]], rows1,
                                  sem1).wait()
            pltpu.sync_copy(rows1, acc_sh.at[didx.at[j0 + 1]], add=True)
            return 0
        lax.fori_loop(0, HALF // 2, body, 0)
    plsc.subcore_barrier()

    pltpu.sync_copy(acc_sh.at[pl.ds(s * NPT, NPT)],
                    p_hbm.at[pl.ds(c * N + s * NPT, NPT)])

    @pl.when(s == 0)
    def _():
        pltpu.sync_copy(acc_sh.at[pl.ds(NS * NPT, TAIL)],
                        p_hbm.at[pl.ds(c * N + NS * NPT, TAIL)])


_agg_call = functools.partial(
    pl.kernel,
    out_type=jax.ShapeDtypeStruct((NC * N, F), jnp.float32),
    mesh=_mesh,
    scratch_types=[
        pltpu.VMEM((HALF, K), jnp.int32),
        pltpu.VMEM((HALF, K), jnp.int32),
        pltpu.VMEM((K, F), jnp.float32),
        pltpu.VMEM((K, F), jnp.float32),
        pltpu.SemaphoreType.DMA,
        pltpu.SemaphoreType.DMA,
        pltpu.VMEM_SHARED((N, F), jnp.float32),
    ],
)(_agg_body)


def _fin_body(p_ref, hs_ref, dg_ref, b_ref, o_ref):
    d = dg_ref[0] + dg_ref[1]
    r = lax.rsqrt(d)
    acc = p_ref[0] + p_ref[1] - hs_ref[...]
    o_ref[...] = jnp.maximum(acc * r + b_ref[...], 0.0)


_RB = 1000
_GRID = N // _RB


def kernel(node_feats, adj, w, b):
    src2d = adj[0].reshape(ROWS, K)
    dst2d = adj[1].reshape(ROWS, K)
    iota = jnp.arange(HR, dtype=jnp.int32).reshape(HR // K, K)

    degw = _deg_call(adj[0], iota)
    deg3 = degw.reshape(NC, N, 1)

    hs = pl.pallas_call(
        _mm_body,
        grid=(_GRID,),
        in_specs=[
            pl.BlockSpec((_RB, F), lambda i: (i, 0)),
            pl.BlockSpec((F, F), lambda i: (0, 0)),
            pl.BlockSpec((NC, _RB, 1), lambda i: (0, i, 0)),
        ],
        out_specs=pl.BlockSpec((_RB, F), lambda i: (i, 0)),
        out_shape=jax.ShapeDtypeStruct((N, F), jnp.float32),
    )(node_feats, w, deg3)

    p = _agg_call(hs, src2d, dst2d).reshape(NC, N, F)

    out = pl.pallas_call(
        _fin_body,
        grid=(_GRID,),
        in_specs=[
            pl.BlockSpec((NC, _RB, F), lambda i: (0, i, 0)),
            pl.BlockSpec((_RB, F), lambda i: (i, 0)),
            pl.BlockSpec((NC, _RB, 1), lambda i: (0, i, 0)),
            pl.BlockSpec((1, F), lambda i: (0, 0)),
        ],
        out_specs=pl.BlockSpec((_RB, F), lambda i: (i, 0)),
        out_shape=jax.ShapeDtypeStruct((N, F), jnp.float32),
    )(p, hs, deg3, b.reshape(1, F))
    return out


# speedup vs baseline: 1.0329x; 1.0329x over previous
import functools

import jax
import jax.numpy as jnp
from jax import lax
from jax.experimental import pallas as pl
from jax.experimental.pallas import tpu as pltpu
from jax.experimental.pallas import tpu_sc as plsc

N = 10000
E = 320000
F = 128
NC = 2
NS = 16
NW = NC * NS
K = 125
HALF = 40
ROWS = E // K
CPW = ROWS // NW
NPT = 624
TAIL = N - NS * NPT
DW = 16

_mesh = plsc.VectorSubcoreMesh(core_axis_name="c", subcore_axis_name="s")


def _deg_body(src_hbm, deg_hbm, idx_v, buf_v, acc_sh):
    c = lax.axis_index("c")
    s = lax.axis_index("s")
    wid = c * NS + s

    pltpu.sync_copy(src_hbm.at[pl.ds(wid * CPW, CPW)], idx_v)

    def zero_row(i, _):
        buf_v[i, :] = jnp.zeros((DW,), jnp.float32)
        return 0
    lax.fori_loop(0, 128, zero_row, 0)
    for z in range(6):
        pltpu.sync_copy(buf_v.at[pl.ds(0, 104)],
                        acc_sh.at[pl.ds(s * NPT + z * 104, 104)])

    @pl.when(s == 0)
    def _():
        pltpu.sync_copy(buf_v.at[pl.ds(0, TAIL)],
                        acc_sh.at[pl.ds(NS * NPT, TAIL)])
    plsc.subcore_barrier()

    def one_row(i, _):
        buf_v[i, :] = jnp.ones((DW,), jnp.float32)
        return 0
    lax.fori_loop(0, 128, one_row, 0)

    def body(j, _):
        pltpu.sync_copy(buf_v.at[pl.ds(0, K)], acc_sh.at[idx_v.at[j]],
                        add=True)
        return 0
    lax.fori_loop(0, CPW, body, 0)
    plsc.subcore_barrier()

    pltpu.sync_copy(acc_sh.at[pl.ds(s * NPT, NPT)],
                    deg_hbm.at[pl.ds(c * N + s * NPT, NPT)])

    @pl.when(s == 0)
    def _():
        pltpu.sync_copy(acc_sh.at[pl.ds(NS * NPT, TAIL)],
                        deg_hbm.at[pl.ds(c * N + NS * NPT, TAIL)])


_deg_call = functools.partial(
    pl.kernel,
    out_type=jax.ShapeDtypeStruct((NC * N, DW), jnp.float32),
    mesh=_mesh,
    scratch_types=[
        pltpu.VMEM((CPW, K), jnp.int32),
        pltpu.VMEM((128, DW), jnp.float32),
        pltpu.VMEM_SHARED((N, DW), jnp.float32),
    ],
)(_deg_body)


def _mm_body(x_ref, w_ref, dg_ref, hs_ref):
    h = jnp.dot(x_ref[...], w_ref[...], preferred_element_type=jnp.float32)
    d = dg_ref[0, :, 0:1] + dg_ref[1, :, 0:1]
    hs_ref[...] = h * lax.rsqrt(d)


def _agg_body(hs_hbm, src_hbm, dst_hbm, p_hbm, sidx, didx, rows0, rows1,
              sem0, sem1, sem2, acc_sh):
    c = lax.axis_index("c")
    s = lax.axis_index("s")
    wid = c * NS + s
    row0 = wid * CPW

    a_s = pltpu.async_copy(src_hbm.at[pl.ds(row0, HALF)], sidx, sem0)
    a_d = pltpu.async_copy(dst_hbm.at[pl.ds(row0, HALF)], didx, sem1)
    a_i = pltpu.async_copy(hs_hbm.at[pl.ds(s * NPT, NPT)],
                           acc_sh.at[pl.ds(s * NPT, NPT)], sem2)

    @pl.when(s == 0)
    def _():
        pltpu.sync_copy(hs_hbm.at[pl.ds(NS * NPT, TAIL)],
                        acc_sh.at[pl.ds(NS * NPT, TAIL)])
    a_s.wait()
    a_d.wait()
    a_i.wait()
    plsc.subcore_barrier()

    for h in range(CPW // HALF):
        if h > 0:
            pltpu.sync_copy(src_hbm.at[pl.ds(row0 + h * HALF, HALF)], sidx)
            pltpu.sync_copy(dst_hbm.at[pl.ds(row0 + h * HALF, HALF)], didx)
        pltpu.async_copy(hs_hbm.at[sidx.at[0]], rows0, sem0)

        def body(i, _):
            j0 = 2 * i
            pltpu.async_copy(hs_hbm.at[sidx.at[j0 + 1]], rows1, sem1)
            pltpu.make_async_copy(hs_hbm.at[sidx.at[j0]], rows0, sem0).wait()
            pltpu.sync_copy(rows0, acc_sh.at[didx.at[j0]], add=True)

            @pl.when(j0 + 2 < HALF)
            def _():
                pltpu.async_copy(hs_hbm.at[sidx.at[j0 + 2]], rows0, sem0)
            pltpu.make_async_copy(hs_hbm.at[sidx.at[j0 + 1]], rows1,
                                  sem1).wait()
            pltpu.sync_copy(rows1, acc_sh.at[didx.at[j0 + 1]], add=True)
            return 0
        lax.fori_loop(0, HALF // 2, body, 0)
    plsc.subcore_barrier()

    pltpu.sync_copy(acc_sh.at[pl.ds(s * NPT, NPT)],
                    p_hbm.at[pl.ds(c * N + s * NPT, NPT)])

    @pl.when(s == 0)
    def _():
        pltpu.sync_copy(acc_sh.at[pl.ds(NS * NPT, TAIL)],
                        p_hbm.at[pl.ds(c * N + NS * NPT, TAIL)])


_agg_call = functools.partial(
    pl.kernel,
    out_type=jax.ShapeDtypeStruct((NC * N, F), jnp.float32),
    mesh=_mesh,
    scratch_types=[
        pltpu.VMEM((HALF, K), jnp.int32),
        pltpu.VMEM((HALF, K), jnp.int32),
        pltpu.VMEM((K, F), jnp.float32),
        pltpu.VMEM((K, F), jnp.float32),
        pltpu.SemaphoreType.DMA,
        pltpu.SemaphoreType.DMA,
        pltpu.SemaphoreType.DMA,
        pltpu.VMEM_SHARED((N, F), jnp.float32),
    ],
)(_agg_body)


def _fin_body(p_ref, hs_ref, dg_ref, b_ref, o_ref):
    d = dg_ref[0, :, 0:1] + dg_ref[1, :, 0:1]
    r = lax.rsqrt(d)
    acc = p_ref[0] + p_ref[1] - hs_ref[...]
    o_ref[...] = jnp.maximum(acc * r + b_ref[...], 0.0)


_RB = 1000
_GRID = N // _RB


def kernel(node_feats, adj, w, b):
    src2d = adj[0].reshape(ROWS, K)
    dst2d = adj[1].reshape(ROWS, K)

    degw = _deg_call(src2d)
    deg3 = degw.reshape(NC, N, DW)

    hs = pl.pallas_call(
        _mm_body,
        grid=(_GRID,),
        in_specs=[
            pl.BlockSpec((_RB, F), lambda i: (i, 0)),
            pl.BlockSpec((F, F), lambda i: (0, 0)),
            pl.BlockSpec((NC, _RB, DW), lambda i: (0, i, 0)),
        ],
        out_specs=pl.BlockSpec((_RB, F), lambda i: (i, 0)),
        out_shape=jax.ShapeDtypeStruct((N, F), jnp.float32),
    )(node_feats, w, deg3)

    p = _agg_call(hs, src2d, dst2d).reshape(NC, N, F)

    out = pl.pallas_call(
        _fin_body,
        grid=(_GRID,),
        in_specs=[
            pl.BlockSpec((NC, _RB, F), lambda i: (0, i, 0)),
            pl.BlockSpec((_RB, F), lambda i: (i, 0)),
            pl.BlockSpec((NC, _RB, DW), lambda i: (0, i, 0)),
            pl.BlockSpec((1, F), lambda i: (0, 0)),
        ],
        out_specs=pl.BlockSpec((_RB, F), lambda i: (i, 0)),
        out_shape=jax.ShapeDtypeStruct((N, F), jnp.float32),
    )(p, hs, deg3, b.reshape(1, F))
    return out


# speedup vs baseline: 1.0548x; 1.0212x over previous
import functools

import jax
import jax.numpy as jnp
from jax import lax
from jax.experimental import pallas as pl
from jax.experimental.pallas import tpu as pltpu
from jax.experimental.pallas import tpu_sc as plsc

N = 10000
E = 320000
F = 128
NC = 2
NS = 16
NW = NC * NS
K = 125
HALF = 40
ROWS = E // K
CPW = ROWS // NW
NPT = 624
TAIL = N - NS * NPT
DW = 16

_mesh = plsc.VectorSubcoreMesh(core_axis_name="c", subcore_axis_name="s")


def _deg_body(src_hbm, deg_hbm, idx_v, buf_v, acc_sh):
    c = lax.axis_index("c")
    s = lax.axis_index("s")
    wid = c * NS + s

    pltpu.sync_copy(src_hbm.at[pl.ds(wid * CPW, CPW)], idx_v)

    def zero_row(i, _):
        buf_v[i, :] = jnp.zeros((DW,), jnp.float32)
        return 0
    lax.fori_loop(0, 128, zero_row, 0)
    for z in range(6):
        pltpu.sync_copy(buf_v.at[pl.ds(0, 104)],
                        acc_sh.at[pl.ds(s * NPT + z * 104, 104)])

    @pl.when(s == 0)
    def _():
        pltpu.sync_copy(buf_v.at[pl.ds(0, TAIL)],
                        acc_sh.at[pl.ds(NS * NPT, TAIL)])
    plsc.subcore_barrier()

    def one_row(i, _):
        buf_v[i, :] = jnp.ones((DW,), jnp.float32)
        return 0
    lax.fori_loop(0, 128, one_row, 0)

    def body(j, _):
        pltpu.sync_copy(buf_v.at[pl.ds(0, K)], acc_sh.at[idx_v.at[j]],
                        add=True)
        return 0
    lax.fori_loop(0, CPW, body, 0)
    plsc.subcore_barrier()

    pltpu.sync_copy(acc_sh.at[pl.ds(s * NPT, NPT)],
                    deg_hbm.at[pl.ds(c * N + s * NPT, NPT)])

    @pl.when(s == 0)
    def _():
        pltpu.sync_copy(acc_sh.at[pl.ds(NS * NPT, TAIL)],
                        deg_hbm.at[pl.ds(c * N + NS * NPT, TAIL)])


_deg_call = functools.partial(
    pl.kernel,
    out_type=jax.ShapeDtypeStruct((NC * N, DW), jnp.float32),
    mesh=_mesh,
    scratch_types=[
        pltpu.VMEM((CPW, K), jnp.int32),
        pltpu.VMEM((128, DW), jnp.float32),
        pltpu.VMEM_SHARED((N, DW), jnp.float32),
    ],
)(_deg_body)


def _mm_body(x_ref, w_ref, dg_ref, hs_ref):
    h = jnp.dot(x_ref[...], w_ref[...], preferred_element_type=jnp.float32)
    d = dg_ref[0, :, 0:1] + dg_ref[1, :, 0:1]
    hs_ref[...] = h * lax.rsqrt(d)


def _agg_body(hs_hbm, src_hbm, dst_hbm, p_hbm, sidx, didx, rows0, rows1,
              sem0, sem1, sem2, acc_sh):
    c = lax.axis_index("c")
    s = lax.axis_index("s")
    wid = c * NS + s
    row0 = wid * CPW

    a_s = pltpu.async_copy(src_hbm.at[pl.ds(row0, HALF)], sidx, sem0)
    a_d = pltpu.async_copy(dst_hbm.at[pl.ds(row0, HALF)], didx, sem1)
    a_i = pltpu.async_copy(hs_hbm.at[pl.ds(s * NPT, NPT)],
                           acc_sh.at[pl.ds(s * NPT, NPT)], sem2)

    @pl.when(s == 0)
    def _():
        pltpu.sync_copy(hs_hbm.at[pl.ds(NS * NPT, TAIL)],
                        acc_sh.at[pl.ds(NS * NPT, TAIL)])
    a_s.wait()
    a_d.wait()
    a_i.wait()
    plsc.subcore_barrier()

    for h in range(CPW // HALF):
        if h > 0:
            pltpu.sync_copy(src_hbm.at[pl.ds(row0 + h * HALF, HALF)], sidx)
            pltpu.sync_copy(dst_hbm.at[pl.ds(row0 + h * HALF, HALF)], didx)
        pltpu.async_copy(hs_hbm.at[sidx.at[0]], rows0, sem0)

        def body(i, _):
            j0 = 2 * i
            pltpu.async_copy(hs_hbm.at[sidx.at[j0 + 1]], rows1, sem1)
            pltpu.make_async_copy(hs_hbm.at[sidx.at[j0]], rows0, sem0).wait()
            pltpu.sync_copy(rows0, acc_sh.at[didx.at[j0]], add=True)

            @pl.when(j0 + 2 < HALF)
            def _():
                pltpu.async_copy(hs_hbm.at[sidx.at[j0 + 2]], rows0, sem0)
            pltpu.make_async_copy(hs_hbm.at[sidx.at[j0 + 1]], rows1,
                                  sem1).wait()
            pltpu.sync_copy(rows1, acc_sh.at[didx.at[j0 + 1]], add=True)
            return 0
        lax.fori_loop(0, HALF // 2, body, 0)
    plsc.subcore_barrier()

    pltpu.sync_copy(acc_sh.at[pl.ds(s * NPT, NPT)],
                    p_hbm.at[pl.ds(c * N + s * NPT, NPT)])

    @pl.when(s == 0)
    def _():
        pltpu.sync_copy(acc_sh.at[pl.ds(NS * NPT, TAIL)],
                        p_hbm.at[pl.ds(c * N + NS * NPT, TAIL)])


_agg_call = functools.partial(
    pl.kernel,
    out_type=jax.ShapeDtypeStruct((NC * N, F), jnp.float32),
    mesh=_mesh,
    scratch_types=[
        pltpu.VMEM((HALF, K), jnp.int32),
        pltpu.VMEM((HALF, K), jnp.int32),
        pltpu.VMEM((K, F), jnp.float32),
        pltpu.VMEM((K, F), jnp.float32),
        pltpu.SemaphoreType.DMA,
        pltpu.SemaphoreType.DMA,
        pltpu.SemaphoreType.DMA,
        pltpu.VMEM_SHARED((N, F), jnp.float32),
    ],
)(_agg_body)


def _fin_body(p_ref, hs_ref, dg_ref, b_ref, o_ref):
    d = dg_ref[0, :, 0:1] + dg_ref[1, :, 0:1]
    r = lax.rsqrt(d)
    acc = p_ref[0] + p_ref[1] - hs_ref[...]
    o_ref[...] = jnp.maximum(acc * r + b_ref[...], 0.0)


_RB = 2000
_GRID = N // _RB


def kernel(node_feats, adj, w, b):
    src2d = adj[0].reshape(ROWS, K)
    dst2d = adj[1].reshape(ROWS, K)

    degw = _deg_call(src2d)
    deg3 = degw.reshape(NC, N, DW)

    hs = pl.pallas_call(
        _mm_body,
        grid=(_GRID,),
        in_specs=[
            pl.BlockSpec((_RB, F), lambda i: (i, 0)),
            pl.BlockSpec((F, F), lambda i: (0, 0)),
            pl.BlockSpec((NC, _RB, DW), lambda i: (0, i, 0)),
        ],
        out_specs=pl.BlockSpec((_RB, F), lambda i: (i, 0)),
        out_shape=jax.ShapeDtypeStruct((N, F), jnp.float32),
    )(node_feats, w, deg3)

    p = _agg_call(hs, src2d, dst2d).reshape(NC, N, F)

    out = pl.pallas_call(
        _fin_body,
        grid=(_GRID,),
        in_specs=[
            pl.BlockSpec((NC, _RB, F), lambda i: (0, i, 0)),
            pl.BlockSpec((_RB, F), lambda i: (i, 0)),
            pl.BlockSpec((NC, _RB, DW), lambda i: (0, i, 0)),
            pl.BlockSpec((1, F), lambda i: (0, 0)),
        ],
        out_specs=pl.BlockSpec((_RB, F), lambda i: (i, 0)),
        out_shape=jax.ShapeDtypeStruct((N, F), jnp.float32),
    )(p, hs, deg3, b.reshape(1, F))
    return out


# speedup vs baseline: 1.0764x; 1.0205x over previous
import functools

import jax
import jax.numpy as jnp
from jax import lax
from jax.experimental import pallas as pl
from jax.experimental.pallas import tpu as pltpu
from jax.experimental.pallas import tpu_sc as plsc

N = 10000
E = 320000
F = 128
NC = 2
NS = 16
NW = NC * NS
K = 125
HALF = 40
ROWS = E // K
CPW = ROWS // NW
NPT = 624
TAIL = N - NS * NPT
DW = 16

_mesh = plsc.VectorSubcoreMesh(core_axis_name="c", subcore_axis_name="s")


def _deg_body(src_hbm, deg_hbm, idx_v, buf_v, sem_d, acc_sh):
    c = lax.axis_index("c")
    s = lax.axis_index("s")
    wid = c * NS + s

    pltpu.sync_copy(src_hbm.at[pl.ds(wid * CPW, CPW)], idx_v)

    def zero_row(i, _):
        buf_v[i, :] = jnp.zeros((DW,), jnp.float32)
        return 0
    lax.fori_loop(0, 128, zero_row, 0)
    for z in range(6):
        pltpu.sync_copy(buf_v.at[pl.ds(0, 104)],
                        acc_sh.at[pl.ds(s * NPT + z * 104, 104)])

    @pl.when(s == 0)
    def _():
        pltpu.sync_copy(buf_v.at[pl.ds(0, TAIL)],
                        acc_sh.at[pl.ds(NS * NPT, TAIL)])
    plsc.subcore_barrier()

    def one_row(i, _):
        buf_v[i, :] = jnp.ones((DW,), jnp.float32)
        return 0
    lax.fori_loop(0, 128, one_row, 0)

    NBUF = 8
    for j in range(NBUF):
        pltpu.async_copy(buf_v.at[pl.ds(0, K)], acc_sh.at[idx_v.at[j]],
                         sem_d, add=True)

    def body(j, _):
        pltpu.make_async_copy(buf_v.at[pl.ds(0, K)],
                              acc_sh.at[idx_v.at[j]], sem_d).wait()
        pltpu.async_copy(buf_v.at[pl.ds(0, K)],
                         acc_sh.at[idx_v.at[j + NBUF]], sem_d, add=True)
        return 0
    lax.fori_loop(0, CPW - NBUF, body, 0)
    for j in range(NBUF):
        pltpu.make_async_copy(buf_v.at[pl.ds(0, K)],
                              acc_sh.at[idx_v.at[j]], sem_d).wait()
    plsc.subcore_barrier()

    pltpu.sync_copy(acc_sh.at[pl.ds(s * NPT, NPT)],
                    deg_hbm.at[pl.ds(c * N + s * NPT, NPT)])

    @pl.when(s == 0)
    def _():
        pltpu.sync_copy(acc_sh.at[pl.ds(NS * NPT, TAIL)],
                        deg_hbm.at[pl.ds(c * N + NS * NPT, TAIL)])


_deg_call = functools.partial(
    pl.kernel,
    out_type=jax.ShapeDtypeStruct((NC * N, DW), jnp.float32),
    mesh=_mesh,
    scratch_types=[
        pltpu.VMEM((CPW, K), jnp.int32),
        pltpu.VMEM((128, DW), jnp.float32),
        pltpu.SemaphoreType.DMA,
        pltpu.VMEM_SHARED((N, DW), jnp.float32),
    ],
)(_deg_body)


def _mm_body(x_ref, w_ref, dg_ref, hs_ref):
    h = jnp.dot(x_ref[...], w_ref[...], preferred_element_type=jnp.float32)
    d = dg_ref[0, :, 0:1] + dg_ref[1, :, 0:1]
    hs_ref[...] = h * lax.rsqrt(d)


def _agg_body(hs_hbm, src_hbm, dst_hbm, p_hbm, sidx, didx, rows0, rows1,
              sem0, sem1, sem2, acc_sh):
    c = lax.axis_index("c")
    s = lax.axis_index("s")
    wid = c * NS + s
    row0 = wid * CPW

    a_s = pltpu.async_copy(src_hbm.at[pl.ds(row0, HALF)], sidx, sem0)
    a_d = pltpu.async_copy(dst_hbm.at[pl.ds(row0, HALF)], didx, sem1)
    a_i = pltpu.async_copy(hs_hbm.at[pl.ds(s * NPT, NPT)],
                           acc_sh.at[pl.ds(s * NPT, NPT)], sem2)

    @pl.when(s == 0)
    def _():
        pltpu.sync_copy(hs_hbm.at[pl.ds(NS * NPT, TAIL)],
                        acc_sh.at[pl.ds(NS * NPT, TAIL)])
    a_s.wait()
    a_d.wait()
    a_i.wait()
    plsc.subcore_barrier()

    for h in range(CPW // HALF):
        if h > 0:
            pltpu.sync_copy(src_hbm.at[pl.ds(row0 + h * HALF, HALF)], sidx)
            pltpu.sync_copy(dst_hbm.at[pl.ds(row0 + h * HALF, HALF)], didx)
        pltpu.async_copy(hs_hbm.at[sidx.at[0]], rows0, sem0)

        def body(i, _):
            j0 = 2 * i
            pltpu.async_copy(hs_hbm.at[sidx.at[j0 + 1]], rows1, sem1)
            pltpu.make_async_copy(hs_hbm.at[sidx.at[j0]], rows0, sem0).wait()
            pltpu.sync_copy(rows0, acc_sh.at[didx.at[j0]], add=True)

            @pl.when(j0 + 2 < HALF)
            def _():
                pltpu.async_copy(hs_hbm.at[sidx.at[j0 + 2]], rows0, sem0)
            pltpu.make_async_copy(hs_hbm.at[sidx.at[j0 + 1]], rows1,
                                  sem1).wait()
            pltpu.sync_copy(rows1, acc_sh.at[didx.at[j0 + 1]], add=True)
            return 0
        lax.fori_loop(0, HALF // 2, body, 0)
    plsc.subcore_barrier()

    pltpu.sync_copy(acc_sh.at[pl.ds(s * NPT, NPT)],
                    p_hbm.at[pl.ds(c * N + s * NPT, NPT)])

    @pl.when(s == 0)
    def _():
        pltpu.sync_copy(acc_sh.at[pl.ds(NS * NPT, TAIL)],
                        p_hbm.at[pl.ds(c * N + NS * NPT, TAIL)])


_agg_call = functools.partial(
    pl.kernel,
    out_type=jax.ShapeDtypeStruct((NC * N, F), jnp.float32),
    mesh=_mesh,
    scratch_types=[
        pltpu.VMEM((HALF, K), jnp.int32),
        pltpu.VMEM((HALF, K), jnp.int32),
        pltpu.VMEM((K, F), jnp.float32),
        pltpu.VMEM((K, F), jnp.float32),
        pltpu.SemaphoreType.DMA,
        pltpu.SemaphoreType.DMA,
        pltpu.SemaphoreType.DMA,
        pltpu.VMEM_SHARED((N, F), jnp.float32),
    ],
)(_agg_body)


def _fin_body(p_ref, hs_ref, dg_ref, b_ref, o_ref):
    d = dg_ref[0, :, 0:1] + dg_ref[1, :, 0:1]
    r = lax.rsqrt(d)
    acc = p_ref[0] + p_ref[1] - hs_ref[...]
    o_ref[...] = jnp.maximum(acc * r + b_ref[...], 0.0)


_RB = 2000
_GRID = N // _RB


def kernel(node_feats, adj, w, b):
    src2d = adj[0].reshape(ROWS, K)
    dst2d = adj[1].reshape(ROWS, K)

    degw = _deg_call(src2d)
    deg3 = degw.reshape(NC, N, DW)

    hs = pl.pallas_call(
        _mm_body,
        grid=(_GRID,),
        in_specs=[
            pl.BlockSpec((_RB, F), lambda i: (i, 0)),
            pl.BlockSpec((F, F), lambda i: (0, 0)),
            pl.BlockSpec((NC, _RB, DW), lambda i: (0, i, 0)),
        ],
        out_specs=pl.BlockSpec((_RB, F), lambda i: (i, 0)),
        out_shape=jax.ShapeDtypeStruct((N, F), jnp.float32),
    )(node_feats, w, deg3)

    p = _agg_call(hs, src2d, dst2d).reshape(NC, N, F)

    out = pl.pallas_call(
        _fin_body,
        grid=(_GRID,),
        in_specs=[
            pl.BlockSpec((NC, _RB, F), lambda i: (0, i, 0)),
            pl.BlockSpec((_RB, F), lambda i: (i, 0)),
            pl.BlockSpec((NC, _RB, DW), lambda i: (0, i, 0)),
            pl.BlockSpec((1, F), lambda i: (0, 0)),
        ],
        out_specs=pl.BlockSpec((_RB, F), lambda i: (i, 0)),
        out_shape=jax.ShapeDtypeStruct((N, F), jnp.float32),
    )(p, hs, deg3, b.reshape(1, F))
    return out


# speedup vs baseline: 1.0774x; 1.0010x over previous
import functools

import jax
import jax.numpy as jnp
from jax import lax
from jax.experimental import pallas as pl
from jax.experimental.pallas import tpu as pltpu
from jax.experimental.pallas import tpu_sc as plsc

N = 10000
E = 320000
F = 128
NC = 2
NS = 16
NW = NC * NS
K = 125
HALF = 40
ROWS = E // K
CPW = ROWS // NW
NPT = 624
TAIL = N - NS * NPT
DW = 16

_mesh = plsc.VectorSubcoreMesh(core_axis_name="c", subcore_axis_name="s")


def _deg_body(src_hbm, deg_hbm, idx_v, buf_v, sem_d, acc_sh):
    c = lax.axis_index("c")
    s = lax.axis_index("s")
    wid = c * NS + s

    pltpu.sync_copy(src_hbm.at[pl.ds(wid * CPW, CPW)], idx_v)

    def zero_row(i, _):
        buf_v[i, :] = jnp.zeros((DW,), jnp.float32)
        return 0
    lax.fori_loop(0, 128, zero_row, 0)
    for z in range(6):
        pltpu.sync_copy(buf_v.at[pl.ds(0, 104)],
                        acc_sh.at[pl.ds(s * NPT + z * 104, 104)])

    @pl.when(s == 0)
    def _():
        pltpu.sync_copy(buf_v.at[pl.ds(0, TAIL)],
                        acc_sh.at[pl.ds(NS * NPT, TAIL)])
    plsc.subcore_barrier()

    def one_row(i, _):
        buf_v[i, :] = jnp.ones((DW,), jnp.float32)
        return 0
    lax.fori_loop(0, 128, one_row, 0)

    NBUF = 16
    for j in range(NBUF):
        pltpu.async_copy(buf_v.at[pl.ds(0, K)], acc_sh.at[idx_v.at[j]],
                         sem_d, add=True)

    def body(j, _):
        pltpu.make_async_copy(buf_v.at[pl.ds(0, K)],
                              acc_sh.at[idx_v.at[j]], sem_d).wait()
        pltpu.async_copy(buf_v.at[pl.ds(0, K)],
                         acc_sh.at[idx_v.at[j + NBUF]], sem_d, add=True)
        return 0
    lax.fori_loop(0, CPW - NBUF, body, 0)
    for j in range(NBUF):
        pltpu.make_async_copy(buf_v.at[pl.ds(0, K)],
                              acc_sh.at[idx_v.at[j]], sem_d).wait()
    plsc.subcore_barrier()

    pltpu.sync_copy(acc_sh.at[pl.ds(s * NPT, NPT)],
                    deg_hbm.at[pl.ds(c * N + s * NPT, NPT)])

    @pl.when(s == 0)
    def _():
        pltpu.sync_copy(acc_sh.at[pl.ds(NS * NPT, TAIL)],
                        deg_hbm.at[pl.ds(c * N + NS * NPT, TAIL)])


_deg_call = functools.partial(
    pl.kernel,
    out_type=jax.ShapeDtypeStruct((NC * N, DW), jnp.float32),
    mesh=_mesh,
    scratch_types=[
        pltpu.VMEM((CPW, K), jnp.int32),
        pltpu.VMEM((128, DW), jnp.float32),
        pltpu.SemaphoreType.DMA,
        pltpu.VMEM_SHARED((N, DW), jnp.float32),
    ],
)(_deg_body)


def _mm_body(x_ref, w_ref, dg_ref, hs_ref):
    h = jnp.dot(x_ref[...], w_ref[...], preferred_element_type=jnp.float32)
    d = dg_ref[0, :, 0:1] + dg_ref[1, :, 0:1]
    hs_ref[...] = h * lax.rsqrt(d)


def _agg_body(hs_hbm, src_hbm, dst_hbm, p_hbm, sidx, didx, rows0, rows1,
              sem0, sem1, sem2, acc_sh):
    c = lax.axis_index("c")
    s = lax.axis_index("s")
    wid = c * NS + s
    row0 = wid * CPW

    a_s = pltpu.async_copy(src_hbm.at[pl.ds(row0, HALF)], sidx, sem0)
    a_d = pltpu.async_copy(dst_hbm.at[pl.ds(row0, HALF)], didx, sem1)
    a_i = pltpu.async_copy(hs_hbm.at[pl.ds(s * NPT, NPT)],
                           acc_sh.at[pl.ds(s * NPT, NPT)], sem2)

    @pl.when(s == 0)
    def _():
        pltpu.sync_copy(hs_hbm.at[pl.ds(NS * NPT, TAIL)],
                        acc_sh.at[pl.ds(NS * NPT, TAIL)])
    a_s.wait()
    a_d.wait()
    a_i.wait()
    plsc.subcore_barrier()

    for h in range(CPW // HALF):
        if h > 0:
            pltpu.sync_copy(src_hbm.at[pl.ds(row0 + h * HALF, HALF)], sidx)
            pltpu.sync_copy(dst_hbm.at[pl.ds(row0 + h * HALF, HALF)], didx)
        pltpu.async_copy(hs_hbm.at[sidx.at[0]], rows0, sem0)

        def body(i, _):
            j0 = 2 * i
            pltpu.async_copy(hs_hbm.at[sidx.at[j0 + 1]], rows1, sem1)
            pltpu.make_async_copy(hs_hbm.at[sidx.at[j0]], rows0, sem0).wait()
            pltpu.sync_copy(rows0, acc_sh.at[didx.at[j0]], add=True)

            @pl.when(j0 + 2 < HALF)
            def _():
                pltpu.async_copy(hs_hbm.at[sidx.at[j0 + 2]], rows0, sem0)
            pltpu.make_async_copy(hs_hbm.at[sidx.at[j0 + 1]], rows1,
                                  sem1).wait()
            pltpu.sync_copy(rows1, acc_sh.at[didx.at[j0 + 1]], add=True)
            return 0
        lax.fori_loop(0, HALF // 2, body, 0)
    plsc.subcore_barrier()

    pltpu.sync_copy(acc_sh.at[pl.ds(s * NPT, NPT)],
                    p_hbm.at[pl.ds(c * N + s * NPT, NPT)])

    @pl.when(s == 0)
    def _():
        pltpu.sync_copy(acc_sh.at[pl.ds(NS * NPT, TAIL)],
                        p_hbm.at[pl.ds(c * N + NS * NPT, TAIL)])


_agg_call = functools.partial(
    pl.kernel,
    out_type=jax.ShapeDtypeStruct((NC * N, F), jnp.float32),
    mesh=_mesh,
    scratch_types=[
        pltpu.VMEM((HALF, K), jnp.int32),
        pltpu.VMEM((HALF, K), jnp.int32),
        pltpu.VMEM((K, F), jnp.float32),
        pltpu.VMEM((K, F), jnp.float32),
        pltpu.SemaphoreType.DMA,
        pltpu.SemaphoreType.DMA,
        pltpu.SemaphoreType.DMA,
        pltpu.VMEM_SHARED((N, F), jnp.float32),
    ],
)(_agg_body)


def _fin_body(p_ref, hs_ref, dg_ref, b_ref, o_ref):
    d = dg_ref[0, :, 0:1] + dg_ref[1, :, 0:1]
    r = lax.rsqrt(d)
    acc = p_ref[0] + p_ref[1] - hs_ref[...]
    o_ref[...] = jnp.maximum(acc * r + b_ref[...], 0.0)


_RB = 2000
_GRID = N // _RB


def kernel(node_feats, adj, w, b):
    src2d = adj[0].reshape(ROWS, K)
    dst2d = adj[1].reshape(ROWS, K)

    degw = _deg_call(src2d)
    deg3 = degw.reshape(NC, N, DW)

    hs = pl.pallas_call(
        _mm_body,
        grid=(_GRID,),
        in_specs=[
            pl.BlockSpec((_RB, F), lambda i: (i, 0)),
            pl.BlockSpec((F, F), lambda i: (0, 0)),
            pl.BlockSpec((NC, _RB, DW), lambda i: (0, i, 0)),
        ],
        out_specs=pl.BlockSpec((_RB, F), lambda i: (i, 0)),
        out_shape=jax.ShapeDtypeStruct((N, F), jnp.float32),
    )(node_feats, w, deg3)

    p = _agg_call(hs, src2d, dst2d).reshape(NC, N, F)

    out = pl.pallas_call(
        _fin_body,
        grid=(_GRID,),
        in_specs=[
            pl.BlockSpec((NC, _RB, F), lambda i: (0, i, 0)),
            pl.BlockSpec((_RB, F), lambda i: (i, 0)),
            pl.BlockSpec((NC, _RB, DW), lambda i: (0, i, 0)),
            pl.BlockSpec((1, F), lambda i: (0, 0)),
        ],
        out_specs=pl.BlockSpec((_RB, F), lambda i: (i, 0)),
        out_shape=jax.ShapeDtypeStruct((N, F), jnp.float32),
    )(p, hs, deg3, b.reshape(1, F))
    return out


# speedup vs baseline: 1.0793x; 1.0018x over previous
import functools

import jax
import jax.numpy as jnp
from jax import lax
from jax.experimental import pallas as pl
from jax.experimental.pallas import tpu as pltpu
from jax.experimental.pallas import tpu_sc as plsc

N = 10000
E = 320000
F = 128
NC = 2
NS = 16
NW = NC * NS
K = 125
HALF = 40
ROWS = E // K
CPW = ROWS // NW
NPT = 624
TAIL = N - NS * NPT
DW = 16

_mesh = plsc.VectorSubcoreMesh(core_axis_name="c", subcore_axis_name="s")


def _deg_body(src_hbm, deg_hbm, idx_v, buf_v, sem_d, acc_sh):
    c = lax.axis_index("c")
    s = lax.axis_index("s")
    wid = c * NS + s

    pltpu.sync_copy(src_hbm.at[pl.ds(wid * CPW, CPW)], idx_v)

    def zero_row(i, _):
        buf_v[i, :] = jnp.zeros((DW,), jnp.float32)
        return 0
    lax.fori_loop(0, 128, zero_row, 0)
    for z in range(6):
        pltpu.sync_copy(buf_v.at[pl.ds(0, 104)],
                        acc_sh.at[pl.ds(s * NPT + z * 104, 104)])

    @pl.when(s == 0)
    def _():
        pltpu.sync_copy(buf_v.at[pl.ds(0, TAIL)],
                        acc_sh.at[pl.ds(NS * NPT, TAIL)])
    plsc.subcore_barrier()

    def one_row(i, _):
        buf_v[i, :] = jnp.ones((DW,), jnp.float32)
        return 0
    lax.fori_loop(0, 128, one_row, 0)

    NBUF = 8
    for j in range(NBUF):
        pltpu.async_copy(buf_v.at[pl.ds(0, K)], acc_sh.at[idx_v.at[j]],
                         sem_d, add=True)

    def body(j, _):
        pltpu.make_async_copy(buf_v.at[pl.ds(0, K)],
                              acc_sh.at[idx_v.at[j]], sem_d).wait()
        pltpu.async_copy(buf_v.at[pl.ds(0, K)],
                         acc_sh.at[idx_v.at[j + NBUF]], sem_d, add=True)
        return 0
    lax.fori_loop(0, CPW - NBUF, body, 0)
    for j in range(NBUF):
        pltpu.make_async_copy(buf_v.at[pl.ds(0, K)],
                              acc_sh.at[idx_v.at[j]], sem_d).wait()
    plsc.subcore_barrier()

    pltpu.sync_copy(acc_sh.at[pl.ds(s * NPT, NPT)],
                    deg_hbm.at[pl.ds(c * N + s * NPT, NPT)])

    @pl.when(s == 0)
    def _():
        pltpu.sync_copy(acc_sh.at[pl.ds(NS * NPT, TAIL)],
                        deg_hbm.at[pl.ds(c * N + NS * NPT, TAIL)])


_deg_call = functools.partial(
    pl.kernel,
    out_type=jax.ShapeDtypeStruct((NC * N, DW), jnp.float32),
    mesh=_mesh,
    scratch_types=[
        pltpu.VMEM((CPW, K), jnp.int32),
        pltpu.VMEM((128, DW), jnp.float32),
        pltpu.SemaphoreType.DMA,
        pltpu.VMEM_SHARED((N, DW), jnp.float32),
    ],
)(_deg_body)


def _mm_body(x_ref, w_ref, dg_ref, hs_ref):
    h = jnp.dot(x_ref[...], w_ref[...], preferred_element_type=jnp.float32)
    d = dg_ref[0, :, 0:1] + dg_ref[1, :, 0:1]
    hs_ref[...] = h * lax.rsqrt(d)


def _agg_body(hs_hbm, src_hbm, dst_hbm, p_hbm, sidx, didx, rows0, rows1,
              sem0, sem1, sem2, acc_sh):
    c = lax.axis_index("c")
    s = lax.axis_index("s")
    wid = c * NS + s
    row0 = wid * CPW

    a_s = pltpu.async_copy(src_hbm.at[pl.ds(row0, HALF)], sidx, sem0)
    a_d = pltpu.async_copy(dst_hbm.at[pl.ds(row0, HALF)], didx, sem1)
    a_i = pltpu.async_copy(hs_hbm.at[pl.ds(s * NPT, NPT)],
                           acc_sh.at[pl.ds(s * NPT, NPT)], sem2)

    @pl.when(s == 0)
    def _():
        pltpu.sync_copy(hs_hbm.at[pl.ds(NS * NPT, TAIL)],
                        acc_sh.at[pl.ds(NS * NPT, TAIL)])
    a_s.wait()
    a_d.wait()
    a_i.wait()
    plsc.subcore_barrier()

    for h in range(CPW // HALF):
        if h > 0:
            pltpu.sync_copy(src_hbm.at[pl.ds(row0 + h * HALF, HALF)], sidx)
            pltpu.sync_copy(dst_hbm.at[pl.ds(row0 + h * HALF, HALF)], didx)
        pltpu.async_copy(hs_hbm.at[sidx.at[0]], rows0, sem0)

        def body(i, _):
            j0 = 2 * i
            pltpu.async_copy(hs_hbm.at[sidx.at[j0 + 1]], rows1, sem1)
            pltpu.make_async_copy(hs_hbm.at[sidx.at[j0]], rows0, sem0).wait()
            pltpu.sync_copy(rows0, acc_sh.at[didx.at[j0]], add=True)

            @pl.when(j0 + 2 < HALF)
            def _():
                pltpu.async_copy(hs_hbm.at[sidx.at[j0 + 2]], rows0, sem0)
            pltpu.make_async_copy(hs_hbm.at[sidx.at[j0 + 1]], rows1,
                                  sem1).wait()
            pltpu.sync_copy(rows1, acc_sh.at[didx.at[j0 + 1]], add=True)
            return 0
        lax.fori_loop(0, HALF // 2, body, 0)
    plsc.subcore_barrier()

    pltpu.sync_copy(acc_sh.at[pl.ds(s * NPT, NPT)],
                    p_hbm.at[pl.ds(c * N + s * NPT, NPT)])

    @pl.when(s == 0)
    def _():
        pltpu.sync_copy(acc_sh.at[pl.ds(NS * NPT, TAIL)],
                        p_hbm.at[pl.ds(c * N + NS * NPT, TAIL)])


_agg_call = functools.partial(
    pl.kernel,
    out_type=jax.ShapeDtypeStruct((NC * N, F), jnp.float32),
    mesh=_mesh,
    scratch_types=[
        pltpu.VMEM((HALF, K), jnp.int32),
        pltpu.VMEM((HALF, K), jnp.int32),
        pltpu.VMEM((K, F), jnp.float32),
        pltpu.VMEM((K, F), jnp.float32),
        pltpu.SemaphoreType.DMA,
        pltpu.SemaphoreType.DMA,
        pltpu.SemaphoreType.DMA,
        pltpu.VMEM_SHARED((N, F), jnp.float32),
    ],
)(_agg_body)


def _fin_body(p_ref, hs_ref, dg_ref, b_ref, o_ref):
    d = dg_ref[0, :, 0:1] + dg_ref[1, :, 0:1]
    r = lax.rsqrt(d)
    acc = p_ref[0] + p_ref[1] - hs_ref[...]
    o_ref[...] = jnp.maximum(acc * r + b_ref[...], 0.0)


_RB = 2000
_GRID = N // _RB


def kernel(node_feats, adj, w, b):
    src2d = adj[0].reshape(ROWS, K)
    dst2d = adj[1].reshape(ROWS, K)

    degw = _deg_call(src2d)
    deg3 = degw.reshape(NC, N, DW)

    hs = pl.pallas_call(
        _mm_body,
        grid=(_GRID,),
        in_specs=[
            pl.BlockSpec((_RB, F), lambda i: (i, 0)),
            pl.BlockSpec((F, F), lambda i: (0, 0)),
            pl.BlockSpec((NC, _RB, DW), lambda i: (0, i, 0)),
        ],
        out_specs=pl.BlockSpec((_RB, F), lambda i: (i, 0)),
        out_shape=jax.ShapeDtypeStruct((N, F), jnp.float32),
    )(node_feats, w, deg3)

    p = _agg_call(hs, src2d, dst2d).reshape(NC, N, F)

    out = pl.pallas_call(
        _fin_body,
        grid=(_GRID,),
        in_specs=[
            pl.BlockSpec((NC, _RB, F), lambda i: (0, i, 0)),
            pl.BlockSpec((_RB, F), lambda i: (i, 0)),
            pl.BlockSpec((NC, _RB, DW), lambda i: (0, i, 0)),
            pl.BlockSpec((1, F), lambda i: (0, 0)),
        ],
        out_specs=pl.BlockSpec((_RB, F), lambda i: (i, 0)),
        out_shape=jax.ShapeDtypeStruct((N, F), jnp.float32),
    )(p, hs, deg3, b.reshape(1, F))
    return out
